# Initial kernel scaffold; baseline (speedup 1.0000x reference)
import jax
import jax.numpy as jnp
from jax.experimental import pallas as pl


def kernel(dept_x, emp_x, dept_edge_index, emp_edge_index, dept_idx, W1, a_src1, a_dst1, b1, W2, a_src2, a_dst2, b2, W3, a_src3, a_dst3, b3, W4, a_src4, a_dst4, b4):
    raise NotImplementedError("write your pallas kernel here")


# speedup vs baseline: 23.8712x; 23.8712x over previous
import functools

import jax
import jax.numpy as jnp
from jax import lax
from jax.experimental import pallas as pl
from jax.experimental.pallas import tpu as pltpu
from jax.experimental.pallas import tpu_sc as plsc

F32 = jnp.float32
I32 = jnp.int32

NC = 2
NS = 16
L = 16
C = 64


def _round_up(x, m):
    return (x + m - 1) // m * m


def _make_gat_sc(n_pad, e_pad, heads, oc, t_split):
    hc = heads // t_split
    feat = hc * oc
    wc = feat + 16
    rpt = n_pad // NS
    assert rpt % C == 0 and n_pad % NS == 0
    if t_split == 2:
        ept = e_pad // NS
    else:
        ept = e_pad // (NS * NC)
    assert ept % C == 0
    n_chunks = ept // C

    mesh = plsc.VectorSubcoreMesh(core_axis_name="c", subcore_axis_name="s", num_cores=NC, num_subcores=NS)

    def body(t0, t1, aux, src, dst, out, acc, rbuf, dbuf, sidx, didx, sem):
        cid = lax.axis_index("c")
        sid = lax.axis_index("s")

        def zrow(i, _):
            for q in range(wc // L):
                rbuf[i, pl.ds(q * L, L)] = jnp.zeros((L,), F32)
            return 0
        lax.fori_loop(0, C, zrow, 0)
        def zcp(z, _):
            pltpu.sync_copy(rbuf, acc.at[pl.ds(sid * rpt + z * C, C)])
            return 0
        lax.fori_loop(0, rpt // C, zcp, 0)
        plsc.subcore_barrier()

        if t_split == 2:
            base0 = sid * ept
        else:
            base0 = (cid * NS + sid) * ept
        adbase = cid * hc if t_split == 2 else 0

        def chunk(g, _):
            base = base0 + g * C
            pltpu.sync_copy(src.at[pl.ds(base, C)], sidx)
            pltpu.sync_copy(dst.at[pl.ds(base, C)], didx)

            @pl.when(cid == 0)
            def _():
                pltpu.async_copy(t0.at[sidx], rbuf, sem).wait()

            @pl.when(cid == 1)
            def _():
                pltpu.async_copy(t1.at[sidx], rbuf, sem).wait()

            pltpu.async_copy(aux.at[didx], dbuf, sem).wait()

            def wgroup(j, _):
                rows = lax.iota(I32, L) + j * L
                for k in range(hc):
                    colr = jnp.full((L,), feat + k, I32)
                    asv = plsc.load_gather(rbuf, [rows, colr])
                    adv = plsc.load_gather(dbuf, [rows, jnp.full((L,), adbase + k, I32)])
                    x = asv + adv
                    w = jnp.exp(jnp.maximum(x, x * jnp.float32(0.2)))
                    plsc.store_scatter(rbuf, [rows, colr], w)
                return 0
            lax.fori_loop(0, C // L, wgroup, 0)

            def edge(e, _):
                for k in range(hc):
                    wk = plsc.load_gather(
                        rbuf, [jnp.full((L,), e, I32), jnp.full((L,), feat + k, I32)])
                    for q in range(oc // L):
                        col = k * oc + q * L
                        rbuf[e, pl.ds(col, L)] = rbuf[e, pl.ds(col, L)] * wk
                return 0
            lax.fori_loop(0, C, edge, 0)

            pltpu.sync_copy(rbuf, acc.at[didx], add=True)
            return 0
        lax.fori_loop(0, n_chunks, chunk, 0)
        plsc.subcore_barrier()

        def wout(z, _):
            r0 = sid * rpt + z * C
            pltpu.sync_copy(acc.at[pl.ds(r0, C)], rbuf)
            pltpu.sync_copy(rbuf, out.at[cid, pl.ds(r0, C)])
            return 0
        lax.fori_loop(0, rpt // C, wout, 0)

    return pl.kernel(
        body,
        out_type=jax.ShapeDtypeStruct((NC, n_pad, wc), F32),
        mesh=mesh,
        compiler_params=pltpu.CompilerParams(use_tc_tiling_on_sc=False, needs_layout_passes=False),
        scratch_types=[
            pltpu.VMEM_SHARED((n_pad, wc), F32),
            pltpu.VMEM((C, wc), F32),
            pltpu.VMEM((C, 16), F32),
            pltpu.VMEM((C,), I32),
            pltpu.VMEM((C,), I32),
            pltpu.SemaphoreType.DMA,
        ],
    )


def _make_gather_rows(n_rows, d, n_out):
    per_w = n_out // (NC * NS)
    assert n_out % (NC * NS) == 0 and per_w % C == 0
    mesh = plsc.VectorSubcoreMesh(core_axis_name="c", subcore_axis_name="s", num_cores=NC, num_subcores=NS)

    def body(tbl, idx, out, iv, rv, sem):
        cid = lax.axis_index("c")
        sid = lax.axis_index("s")
        base0 = (sid * NC + cid) * per_w

        def chunk(z, _):
            b = base0 + z * C
            pltpu.sync_copy(idx.at[pl.ds(b, C)], iv)
            pltpu.async_copy(tbl.at[iv], rv, sem).wait()
            pltpu.sync_copy(rv, out.at[pl.ds(b, C)])
            return 0
        lax.fori_loop(0, per_w // C, chunk, 0)

    return pl.kernel(
        body,
        out_type=jax.ShapeDtypeStruct((n_out, d), F32),
        mesh=mesh,
        compiler_params=pltpu.CompilerParams(
            use_tc_tiling_on_sc=False, needs_layout_passes=False),
        scratch_types=[
            pltpu.VMEM((C,), I32),
            pltpu.VMEM((C, d), F32),
            pltpu.SemaphoreType.DMA,
        ],
    )


def _tc_call(body, out_shapes):
    return pl.pallas_call(
        body, out_shape=[jax.ShapeDtypeStruct(s, F32) for s in out_shapes])


def _tc_proj4(x_pad, wcat, n_pad):
    z14 = (n_pad, 14)

    def body(x_ref, w_ref, t0_ref, t1_ref, aux_ref):
        t = jnp.dot(x_ref[...], w_ref[...], preferred_element_type=F32)
        z = jnp.zeros(z14, F32)
        t0_ref[...] = jnp.concatenate([t[:, 0:128], t[:, 256:258], z], axis=1)
        t1_ref[...] = jnp.concatenate([t[:, 128:256], t[:, 258:260], z], axis=1)
        aux_ref[...] = jnp.concatenate(
            [t[:, 260:264], jnp.zeros((n_pad, 12), F32)], axis=1)

    return _tc_call(body, [(n_pad, 144), (n_pad, 144), (n_pad, 16)])(x_pad, wcat)


def _tc_norm4_proj(acc, b, wcat, n_pad, oc_next, relu):
    wnext = _round_up(oc_next + 1, 16)

    def body(acc_ref, b_ref, w_ref, t_ref, aux_ref):
        num = jnp.concatenate([acc_ref[0, :, 0:128], acc_ref[1, :, 0:128]], axis=1)
        den = jnp.concatenate([acc_ref[0, :, 128:130], acc_ref[1, :, 128:130]], axis=1)
        denr = jnp.broadcast_to(den[:, :, None], (n_pad, 4, 64)).reshape(n_pad, 256)
        h = num / denr + b_ref[...]
        if relu:
            h = jnp.maximum(h, 0.0)
        t = jnp.dot(h, w_ref[...], preferred_element_type=F32)
        t_ref[...] = jnp.concatenate(
            [t[:, 0:oc_next + 1], jnp.zeros((n_pad, wnext - oc_next - 1), F32)], axis=1)
        aux_ref[...] = jnp.concatenate(
            [t[:, oc_next + 1:oc_next + 2], jnp.zeros((n_pad, 15), F32)], axis=1)

    return _tc_call(body, [(n_pad, wnext), (n_pad, 16)])(acc, b.reshape(1, -1), wcat)


def _tc_norm1(acc, b, oc, n_pad):
    def body(acc_ref, b_ref, out_ref):
        num = acc_ref[0, :, 0:oc] + acc_ref[1, :, 0:oc]
        den = acc_ref[0, :, oc:oc + 1] + acc_ref[1, :, oc:oc + 1]
        out_ref[...] = num / den + b_ref[...]

    return _tc_call(body, [(n_pad, oc)])(acc, b.reshape(1, -1))[0]


def _tc_emp_proj4(emp_x_pad, df, wcat, n_pad):
    def body(x_ref, df_ref, wa_ref, wb_ref, t0_ref, t1_ref, aux_ref):
        t = (jnp.dot(x_ref[...], wa_ref[...], preferred_element_type=F32)
             + jnp.dot(df_ref[...], wb_ref[...], preferred_element_type=F32))
        z = jnp.zeros((n_pad, 14), F32)
        t0_ref[...] = jnp.concatenate([t[:, 0:128], t[:, 256:258], z], axis=1)
        t1_ref[...] = jnp.concatenate([t[:, 128:256], t[:, 258:260], z], axis=1)
        aux_ref[...] = jnp.concatenate(
            [t[:, 260:264], jnp.zeros((n_pad, 12), F32)], axis=1)

    return _tc_call(body, [(n_pad, 144), (n_pad, 144), (n_pad, 16)])(
        emp_x_pad, df, wcat[:128], wcat[128:])


def _tc_final(acc, b, n, n_pad):
    def body(acc_ref, b_ref, out_ref):
        num = acc_ref[0, :, 0:32] + acc_ref[1, :, 0:32]
        den = acc_ref[0, :, 32:33] + acc_ref[1, :, 32:33]
        x = num / den + b_ref[...]
        m = jnp.max(x, axis=1, keepdims=True)
        lse = m + jnp.log(jnp.sum(jnp.exp(x - m), axis=1, keepdims=True))
        out_ref[...] = (x - lse)[:n, :]

    return pl.pallas_call(
        body, out_shape=[jax.ShapeDtypeStruct((n, 32), F32)])(
            acc, b.reshape(1, -1))[0]


def _fold(W, a_s, a_d, heads, oc):
    eye = jnp.eye(heads, dtype=F32)
    A_s = (a_s[:, :, None] * eye[:, None, :]).reshape(heads * oc, heads)
    A_d = (a_d[:, :, None] * eye[:, None, :]).reshape(heads * oc, heads)
    return jnp.concatenate([W, W @ A_s, W @ A_d], axis=1)


def _pad_edges(edge_index, n, e_pad):
    src = jnp.concatenate([
        edge_index[0], jnp.arange(n, dtype=I32),
        jnp.full((e_pad - edge_index.shape[1] - n,), n, I32)])
    dst = jnp.concatenate([
        edge_index[1], jnp.arange(n, dtype=I32),
        jnp.full((e_pad - edge_index.shape[1] - n,), n, I32)])
    return src, dst


def kernel(dept_x, emp_x, dept_edge_index, emp_edge_index, dept_idx,
           W1, a_src1, a_dst1, b1, W2, a_src2, a_dst2, b2,
           W3, a_src3, a_dst3, b3, W4, a_src4, a_dst4, b4):
    Dn, En = dept_x.shape[0], emp_x.shape[0]
    Dnp = 1024
    Enp = 10240
    De_pad = _round_up(dept_edge_index.shape[1] + Dn, NS * NC * C)
    Ee_pad = _round_up(emp_edge_index.shape[1] + En, NS * NC * C)

    dsrc, ddst = _pad_edges(dept_edge_index, Dn, De_pad)
    esrc, edst = _pad_edges(emp_edge_index, En, Ee_pad)
    dept_x_pad = jnp.pad(dept_x, ((0, Dnp - Dn), (0, 0)))
    emp_x_pad = jnp.pad(emp_x, ((0, Enp - En), (0, 0)))
    didx_pad = jnp.pad(dept_idx, (0, Enp - En))

    wc1 = _fold(W1, a_src1, a_dst1, 4, 64)
    wc2 = _fold(W2, a_src2, a_dst2, 1, 64)
    wc3 = _fold(W3, a_src3, a_dst3, 4, 64)
    wc4 = _fold(W4, a_src4, a_dst4, 1, 32)

    t0, t1, aux = _tc_proj4(dept_x_pad, wc1, Dnp)
    acc1 = _make_gat_sc(Dnp, De_pad, 4, 64, 2)(t0, t1, aux, dsrc, ddst)
    t2, aux2 = _tc_norm4_proj(acc1, b1, wc2, Dnp, 64, relu=True)
    acc2 = _make_gat_sc(Dnp, De_pad, 1, 64, 1)(t2, t2, aux2, dsrc, ddst)
    d2 = _tc_norm1(acc2, b2, 64, Dnp)
    df = _make_gather_rows(Dnp, 64, Enp)(d2, didx_pad)
    t0, t1, aux = _tc_emp_proj4(emp_x_pad, df, wc3, Enp)
    acc3 = _make_gat_sc(Enp, Ee_pad, 4, 64, 2)(t0, t1, aux, esrc, edst)
    t4, aux4 = _tc_norm4_proj(acc3, b3, wc4, Enp, 32, relu=True)
    acc4 = _make_gat_sc(Enp, Ee_pad, 1, 32, 1)(t4, t4, aux4, esrc, edst)
    return _tc_final(acc4, b4, En, Enp)


# speedup vs baseline: 41.7714x; 1.7499x over previous
import functools

import jax
import jax.numpy as jnp
from jax import lax
from jax.experimental import pallas as pl
from jax.experimental.pallas import tpu as pltpu
from jax.experimental.pallas import tpu_sc as plsc

F32 = jnp.float32
I32 = jnp.int32

NC = 2
NS = 16
L = 16
C = 64


def _round_up(x, m):
    return (x + m - 1) // m * m


def _make_gat_sc(n_pad, e_pad, heads, oc, t_split):
    hc = heads // t_split
    feat = hc * oc
    wc = feat + 16
    rpt = n_pad // NS
    assert rpt % C == 0 and n_pad % NS == 0
    if t_split == 2:
        ept = e_pad // NS
    else:
        ept = e_pad // (NS * NC)
    nloc = ept // C
    assert ept % C == 0 and nloc % 3 == 0 and nloc >= 6

    mesh = plsc.VectorSubcoreMesh(core_axis_name="c", subcore_axis_name="s", num_cores=NC, num_subcores=NS)

    def body(t0, t1, aux, src, dst, out, acc,
             r0, r1, r2, d0, d1, d2, si0, si1, si2, di0, di1, di2,
             sr0, sr1, sr2, sd0, sd1, sd2, ss0, ss1, ss2):
        R = [r0, r1, r2]
        D = [d0, d1, d2]
        SI = [si0, si1, si2]
        DI = [di0, di1, di2]
        SR = [sr0, sr1, sr2]
        SD = [sd0, sd1, sd2]
        SS = [ss0, ss1, ss2]
        cid = lax.axis_index("c")
        sid = lax.axis_index("s")

        def zrow(i, _):
            for q in range(wc // L):
                r0[i, pl.ds(q * L, L)] = jnp.zeros((L,), F32)
            return 0
        lax.fori_loop(0, C, zrow, 0)
        def zcp(z, _):
            pltpu.sync_copy(r0, acc.at[pl.ds(sid * rpt + z * C, C)])
            return 0
        lax.fori_loop(0, rpt // C, zcp, 0)
        plsc.subcore_barrier()

        if t_split == 2:
            base0 = sid * ept
        else:
            base0 = (cid * NS + sid) * ept
        adbase = cid * hc if t_split == 2 else 0

        def issue(g, s):
            base = base0 + g * C
            pltpu.sync_copy(src.at[pl.ds(base, C)], SI[s])
            pltpu.sync_copy(dst.at[pl.ds(base, C)], DI[s])

            @pl.when(cid == 0)
            def _():
                pltpu.async_copy(t0.at[SI[s]], R[s], SR[s])

            @pl.when(cid == 1)
            def _():
                pltpu.async_copy(t1.at[SI[s]], R[s], SR[s])

            pltpu.async_copy(aux.at[DI[s]], D[s], SD[s])

        def wait_gather(s):
            pltpu.make_async_copy(t0.at[SI[s]], R[s], SR[s]).wait()
            pltpu.make_async_copy(aux.at[DI[s]], D[s], SD[s]).wait()

        def scatter(s):
            pltpu.async_copy(R[s], acc.at[DI[s]], SS[s], add=True)

        def wait_scatter(s):
            pltpu.make_async_copy(R[s], acc.at[DI[s]], SS[s]).wait()

        def compute(s):
            rbuf, dbuf = R[s], D[s]

            def wgroup(j, _):
                rows = lax.iota(I32, L) + j * L
                for k in range(hc):
                    colr = jnp.full((L,), feat + k, I32)
                    asv = plsc.load_gather(rbuf, [rows, colr])
                    adv = plsc.load_gather(dbuf, [rows, jnp.full((L,), adbase + k, I32)])
                    x = asv + adv
                    w = jnp.exp(jnp.maximum(x, x * jnp.float32(0.2)))
                    plsc.store_scatter(rbuf, [rows, colr], w)
                return 0
            lax.fori_loop(0, C // L, wgroup, 0)

            def edge(e, _):
                for k in range(hc):
                    wk = plsc.load_gather(
                        rbuf, [jnp.full((L,), e, I32), jnp.full((L,), feat + k, I32)])
                    for q in range(oc // L):
                        col = k * oc + q * L
                        rbuf[e, pl.ds(col, L)] = rbuf[e, pl.ds(col, L)] * wk
                return 0
            lax.fori_loop(0, C, edge, 0)

        def step(g, s, first):
            if not first:
                wait_scatter((s + 1) % 3)
            issue(g + 1, (s + 1) % 3)
            wait_gather(s)
            compute(s)
            scatter(s)

        issue(0, 0)
        step(0, 0, True)
        step(1, 1, True)
        step(2, 2, False)

        def trip(p, _):
            g = p * 3
            step(g, 0, False)
            step(g + 1, 1, False)
            step(g + 2, 2, False)
            return 0
        lax.fori_loop(1, nloc // 3, trip, 0)

        wait_scatter(1)
        wait_scatter(2)
        wait_gather(0)
        plsc.subcore_barrier()

        def wout(z, _):
            rr = sid * rpt + z * C
            pltpu.sync_copy(acc.at[pl.ds(rr, C)], r0)
            pltpu.sync_copy(r0, out.at[cid, pl.ds(rr, C)])
            return 0
        lax.fori_loop(0, rpt // C, wout, 0)

    return pl.kernel(
        body,
        out_type=jax.ShapeDtypeStruct((NC, n_pad, wc), F32),
        mesh=mesh,
        compiler_params=pltpu.CompilerParams(use_tc_tiling_on_sc=False, needs_layout_passes=False),
        scratch_types=(
            [pltpu.VMEM_SHARED((n_pad, wc), F32)]
            + [pltpu.VMEM((C, wc), F32)] * 3
            + [pltpu.VMEM((C, 16), F32)] * 3
            + [pltpu.VMEM((C,), I32)] * 6
            + [pltpu.SemaphoreType.DMA] * 9
        ),
    )


def _make_gather_rows(n_rows, d, n_out):
    per_w = n_out // (NC * NS)
    assert n_out % (NC * NS) == 0 and per_w % C == 0
    mesh = plsc.VectorSubcoreMesh(core_axis_name="c", subcore_axis_name="s", num_cores=NC, num_subcores=NS)

    def body(tbl, idx, out, iv, rv, sem):
        cid = lax.axis_index("c")
        sid = lax.axis_index("s")
        base0 = (sid * NC + cid) * per_w

        def chunk(z, _):
            b = base0 + z * C
            pltpu.sync_copy(idx.at[pl.ds(b, C)], iv)
            pltpu.async_copy(tbl.at[iv], rv, sem).wait()
            pltpu.sync_copy(rv, out.at[pl.ds(b, C)])
            return 0
        lax.fori_loop(0, per_w // C, chunk, 0)

    return pl.kernel(
        body,
        out_type=jax.ShapeDtypeStruct((n_out, d), F32),
        mesh=mesh,
        compiler_params=pltpu.CompilerParams(
            use_tc_tiling_on_sc=False, needs_layout_passes=False),
        scratch_types=[
            pltpu.VMEM((C,), I32),
            pltpu.VMEM((C, d), F32),
            pltpu.SemaphoreType.DMA,
        ],
    )


def _tc_call(body, out_shapes):
    return pl.pallas_call(
        body, out_shape=[jax.ShapeDtypeStruct(s, F32) for s in out_shapes])


def _tc_proj4(x_pad, wcat, n_pad):
    z14 = (n_pad, 14)

    def body(x_ref, w_ref, t0_ref, t1_ref, aux_ref):
        t = jnp.dot(x_ref[...], w_ref[...], preferred_element_type=F32)
        z = jnp.zeros(z14, F32)
        t0_ref[...] = jnp.concatenate([t[:, 0:128], t[:, 256:258], z], axis=1)
        t1_ref[...] = jnp.concatenate([t[:, 128:256], t[:, 258:260], z], axis=1)
        aux_ref[...] = jnp.concatenate(
            [t[:, 260:264], jnp.zeros((n_pad, 12), F32)], axis=1)

    return _tc_call(body, [(n_pad, 144), (n_pad, 144), (n_pad, 16)])(x_pad, wcat)


def _tc_norm4_proj(acc, b, wcat, n_pad, oc_next, relu):
    wnext = _round_up(oc_next + 1, 16)

    def body(acc_ref, b_ref, w_ref, t_ref, aux_ref):
        num = jnp.concatenate([acc_ref[0, :, 0:128], acc_ref[1, :, 0:128]], axis=1)
        den = jnp.concatenate([acc_ref[0, :, 128:130], acc_ref[1, :, 128:130]], axis=1)
        denr = jnp.broadcast_to(den[:, :, None], (n_pad, 4, 64)).reshape(n_pad, 256)
        h = num / denr + b_ref[...]
        if relu:
            h = jnp.maximum(h, 0.0)
        t = jnp.dot(h, w_ref[...], preferred_element_type=F32)
        t_ref[...] = jnp.concatenate(
            [t[:, 0:oc_next + 1], jnp.zeros((n_pad, wnext - oc_next - 1), F32)], axis=1)
        aux_ref[...] = jnp.concatenate(
            [t[:, oc_next + 1:oc_next + 2], jnp.zeros((n_pad, 15), F32)], axis=1)

    return _tc_call(body, [(n_pad, wnext), (n_pad, 16)])(acc, b.reshape(1, -1), wcat)


def _tc_norm1(acc, b, oc, n_pad):
    def body(acc_ref, b_ref, out_ref):
        num = acc_ref[0, :, 0:oc] + acc_ref[1, :, 0:oc]
        den = acc_ref[0, :, oc:oc + 1] + acc_ref[1, :, oc:oc + 1]
        out_ref[...] = num / den + b_ref[...]

    return _tc_call(body, [(n_pad, oc)])(acc, b.reshape(1, -1))[0]


def _tc_emp_proj4(emp_x_pad, df, wcat, n_pad):
    def body(x_ref, df_ref, wa_ref, wb_ref, t0_ref, t1_ref, aux_ref):
        t = (jnp.dot(x_ref[...], wa_ref[...], preferred_element_type=F32)
             + jnp.dot(df_ref[...], wb_ref[...], preferred_element_type=F32))
        z = jnp.zeros((n_pad, 14), F32)
        t0_ref[...] = jnp.concatenate([t[:, 0:128], t[:, 256:258], z], axis=1)
        t1_ref[...] = jnp.concatenate([t[:, 128:256], t[:, 258:260], z], axis=1)
        aux_ref[...] = jnp.concatenate(
            [t[:, 260:264], jnp.zeros((n_pad, 12), F32)], axis=1)

    return _tc_call(body, [(n_pad, 144), (n_pad, 144), (n_pad, 16)])(
        emp_x_pad, df, wcat[:128], wcat[128:])


def _tc_final(acc, b, n, n_pad):
    def body(acc_ref, b_ref, out_ref):
        num = acc_ref[0, :, 0:32] + acc_ref[1, :, 0:32]
        den = acc_ref[0, :, 32:33] + acc_ref[1, :, 32:33]
        x = num / den + b_ref[...]
        m = jnp.max(x, axis=1, keepdims=True)
        lse = m + jnp.log(jnp.sum(jnp.exp(x - m), axis=1, keepdims=True))
        out_ref[...] = (x - lse)[:n, :]

    return pl.pallas_call(
        body, out_shape=[jax.ShapeDtypeStruct((n, 32), F32)])(
            acc, b.reshape(1, -1))[0]


def _fold(W, a_s, a_d, heads, oc):
    eye = jnp.eye(heads, dtype=F32)
    A_s = (a_s[:, :, None] * eye[:, None, :]).reshape(heads * oc, heads)
    A_d = (a_d[:, :, None] * eye[:, None, :]).reshape(heads * oc, heads)
    return jnp.concatenate([W, W @ A_s, W @ A_d], axis=1)


def _pad_edges(edge_index, n, e_pad):
    src = jnp.concatenate([
        edge_index[0], jnp.arange(n, dtype=I32),
        jnp.full((e_pad + C - edge_index.shape[1] - n,), n, I32)])
    dst = jnp.concatenate([
        edge_index[1], jnp.arange(n, dtype=I32),
        jnp.full((e_pad + C - edge_index.shape[1] - n,), n, I32)])
    return src, dst


def kernel(dept_x, emp_x, dept_edge_index, emp_edge_index, dept_idx,
           W1, a_src1, a_dst1, b1, W2, a_src2, a_dst2, b2,
           W3, a_src3, a_dst3, b3, W4, a_src4, a_dst4, b4):
    Dn, En = dept_x.shape[0], emp_x.shape[0]
    Dnp = 1024
    Enp = 10240
    De_pad = _round_up(dept_edge_index.shape[1] + Dn, NS * NC * C * 3)
    Ee_pad = _round_up(emp_edge_index.shape[1] + En, NS * NC * C * 3)

    dsrc, ddst = _pad_edges(dept_edge_index, Dn, De_pad)
    esrc, edst = _pad_edges(emp_edge_index, En, Ee_pad)
    dept_x_pad = jnp.pad(dept_x, ((0, Dnp - Dn), (0, 0)))
    emp_x_pad = jnp.pad(emp_x, ((0, Enp - En), (0, 0)))
    didx_pad = jnp.pad(dept_idx, (0, Enp - En))

    wc1 = _fold(W1, a_src1, a_dst1, 4, 64)
    wc2 = _fold(W2, a_src2, a_dst2, 1, 64)
    wc3 = _fold(W3, a_src3, a_dst3, 4, 64)
    wc4 = _fold(W4, a_src4, a_dst4, 1, 32)

    t0, t1, aux = _tc_proj4(dept_x_pad, wc1, Dnp)
    acc1 = _make_gat_sc(Dnp, De_pad, 4, 64, 2)(t0, t1, aux, dsrc, ddst)
    t2, aux2 = _tc_norm4_proj(acc1, b1, wc2, Dnp, 64, relu=True)
    acc2 = _make_gat_sc(Dnp, De_pad, 1, 64, 1)(t2, t2, aux2, dsrc, ddst)
    d2 = _tc_norm1(acc2, b2, 64, Dnp)
    df = _make_gather_rows(Dnp, 64, Enp)(d2, didx_pad)
    t0, t1, aux = _tc_emp_proj4(emp_x_pad, df, wc3, Enp)
    acc3 = _make_gat_sc(Enp, Ee_pad, 4, 64, 2)(t0, t1, aux, esrc, edst)
    t4, aux4 = _tc_norm4_proj(acc3, b3, wc4, Enp, 32, relu=True)
    acc4 = _make_gat_sc(Enp, Ee_pad, 1, 32, 1)(t4, t4, aux4, esrc, edst)
    return _tc_final(acc4, b4, En, Enp)


# speedup vs baseline: 50.5588x; 1.2104x over previous
import functools

import jax
import jax.numpy as jnp
from jax import lax
from jax.experimental import pallas as pl
from jax.experimental.pallas import tpu as pltpu
from jax.experimental.pallas import tpu_sc as plsc

F32 = jnp.float32
I32 = jnp.int32

NC = 2
NS = 16
L = 16
C = 64


def _round_up(x, m):
    return (x + m - 1) // m * m


def _make_gat_sc(n_pad, e_pad, heads, oc, t_split, c):
    hc = heads // t_split
    feat = hc * oc
    wc = feat + 8
    zoffs = list(range(0, wc - 15, L)) + ([wc - L] if wc % L else [])
    rpt = n_pad // NS
    assert rpt % c == 0 and n_pad % NS == 0
    if t_split == 2:
        ept = e_pad // NS
    else:
        ept = e_pad // (NS * NC)
    nloc = ept // c
    assert ept % c == 0 and nloc % 3 == 0 and nloc >= 6

    mesh = plsc.VectorSubcoreMesh(core_axis_name="c", subcore_axis_name="s", num_cores=NC, num_subcores=NS)

    def body(t0, t1, aux, edges, out, acc,
             r0, r1, r2, d0, d1, d2, i0, i1, i2,
             sr0, sr1, sr2, sd0, sd1, sd2, ss0, ss1, ss2):
        R = [r0, r1, r2]
        D = [d0, d1, d2]
        IB = [i0, i1, i2]
        SR = [sr0, sr1, sr2]
        SD = [sd0, sd1, sd2]
        SS = [ss0, ss1, ss2]
        cid = lax.axis_index("c")
        sid = lax.axis_index("s")

        def zrow(i, _):
            for q in zoffs:
                r0[i, pl.ds(q, L)] = jnp.zeros((L,), F32)
            return 0
        lax.fori_loop(0, c, zrow, 0)
        def zcp(z, _):
            pltpu.sync_copy(r0, acc.at[pl.ds(sid * rpt + z * c, c)])
            return 0
        lax.fori_loop(0, rpt // c, zcp, 0)
        plsc.subcore_barrier()

        if t_split == 2:
            base0 = sid * ept
        else:
            base0 = (cid * NS + sid) * ept
        adbase = cid * hc if t_split == 2 else 0

        def issue(g, s):
            base = base0 + g * c
            pltpu.sync_copy(edges.at[:, pl.ds(base, c)], IB[s])

            @pl.when(cid == 0)
            def _():
                pltpu.async_copy(t0.at[IB[s].at[0]], R[s], SR[s])

            @pl.when(cid == 1)
            def _():
                pltpu.async_copy(t1.at[IB[s].at[0]], R[s], SR[s])

            pltpu.async_copy(aux.at[IB[s].at[1]], D[s], SD[s])

        def wait_gather(s):
            pltpu.make_async_copy(t0.at[IB[s].at[0]], R[s], SR[s]).wait()
            pltpu.make_async_copy(aux.at[IB[s].at[1]], D[s], SD[s]).wait()

        def scatter(s):
            pltpu.async_copy(R[s], acc.at[IB[s].at[1]], SS[s], add=True)

        def wait_scatter(s):
            pltpu.make_async_copy(R[s], acc.at[IB[s].at[1]], SS[s]).wait()

        def compute(s):
            rbuf, dbuf = R[s], D[s]

            def wgroup(j, _):
                rows = lax.iota(I32, L) + j * L
                for k in range(hc):
                    colr = jnp.full((L,), feat + k, I32)
                    asv = plsc.load_gather(rbuf, [rows, colr])
                    adv = plsc.load_gather(dbuf, [rows, jnp.full((L,), adbase + k, I32)])
                    x = asv + adv
                    w = jnp.exp(jnp.maximum(x, x * jnp.float32(0.2)))
                    plsc.store_scatter(rbuf, [rows, colr], w)
                return 0
            lax.fori_loop(0, c // L, wgroup, 0)

            def edge(e, _):
                for k in range(hc):
                    wk = plsc.load_gather(
                        rbuf, [jnp.full((L,), e, I32), jnp.full((L,), feat + k, I32)])
                    for q in range(oc // L):
                        col = k * oc + q * L
                        rbuf[e, pl.ds(col, L)] = rbuf[e, pl.ds(col, L)] * wk
                return 0
            lax.fori_loop(0, c, edge, 0)

        def step(g, s, first):
            if not first:
                wait_scatter((s + 1) % 3)
            issue(g + 1, (s + 1) % 3)
            wait_gather(s)
            compute(s)
            scatter(s)

        issue(0, 0)
        step(0, 0, True)
        step(1, 1, True)
        step(2, 2, False)

        def trip(p, _):
            g = p * 3
            step(g, 0, False)
            step(g + 1, 1, False)
            step(g + 2, 2, False)
            return 0
        lax.fori_loop(1, nloc // 3, trip, 0)

        wait_scatter(1)
        wait_scatter(2)
        wait_gather(0)
        plsc.subcore_barrier()

        def wout(z, _):
            rr = sid * rpt + z * c
            pltpu.sync_copy(acc.at[pl.ds(rr, c)], r0)
            pltpu.sync_copy(r0, out.at[cid, pl.ds(rr, c)])
            return 0
        lax.fori_loop(0, rpt // c, wout, 0)

    return pl.kernel(
        body,
        out_type=jax.ShapeDtypeStruct((NC, n_pad, wc), F32),
        mesh=mesh,
        compiler_params=pltpu.CompilerParams(use_tc_tiling_on_sc=False, needs_layout_passes=False),
        scratch_types=(
            [pltpu.VMEM_SHARED((n_pad, wc), F32)]
            + [pltpu.VMEM((c, wc), F32)] * 3
            + [pltpu.VMEM((c, 16), F32)] * 3
            + [pltpu.VMEM((2, c), I32)] * 3
            + [pltpu.SemaphoreType.DMA] * 9
        ),
    )


def _make_gather_rows(n_rows, d, n_out):
    per_w = n_out // (NC * NS)
    assert n_out % (NC * NS) == 0 and per_w % C == 0
    mesh = plsc.VectorSubcoreMesh(core_axis_name="c", subcore_axis_name="s", num_cores=NC, num_subcores=NS)

    def body(tbl, idx, out, iv, rv, sem):
        cid = lax.axis_index("c")
        sid = lax.axis_index("s")
        base0 = (sid * NC + cid) * per_w

        def chunk(z, _):
            b = base0 + z * C
            pltpu.sync_copy(idx.at[pl.ds(b, C)], iv)
            pltpu.async_copy(tbl.at[iv], rv, sem).wait()
            pltpu.sync_copy(rv, out.at[pl.ds(b, C)])
            return 0
        lax.fori_loop(0, per_w // C, chunk, 0)

    return pl.kernel(
        body,
        out_type=jax.ShapeDtypeStruct((n_out, d), F32),
        mesh=mesh,
        compiler_params=pltpu.CompilerParams(
            use_tc_tiling_on_sc=False, needs_layout_passes=False),
        scratch_types=[
            pltpu.VMEM((C,), I32),
            pltpu.VMEM((C, d), F32),
            pltpu.SemaphoreType.DMA,
        ],
    )


def _tc_call(body, out_shapes):
    return pl.pallas_call(
        body, out_shape=[jax.ShapeDtypeStruct(s, F32) for s in out_shapes])


def _tc_proj4(x_pad, wcat, n_pad):
    def body(x_ref, w_ref, t0_ref, t1_ref, aux_ref):
        t = jnp.dot(x_ref[...], w_ref[...], preferred_element_type=F32)
        z = jnp.zeros((n_pad, 6), F32)
        t0_ref[...] = jnp.concatenate([t[:, 0:128], t[:, 256:258], z], axis=1)
        t1_ref[...] = jnp.concatenate([t[:, 128:256], t[:, 258:260], z], axis=1)
        aux_ref[...] = jnp.concatenate(
            [t[:, 260:264], jnp.zeros((n_pad, 12), F32)], axis=1)

    return _tc_call(body, [(n_pad, 136), (n_pad, 136), (n_pad, 16)])(x_pad, wcat)


def _tc_norm4_proj(acc, b, wcat, n_pad, oc_next, relu):
    wnext = _round_up(oc_next + 1, 8)

    def body(acc_ref, b_ref, w_ref, t_ref, aux_ref):
        num = jnp.concatenate([acc_ref[0, :, 0:128], acc_ref[1, :, 0:128]], axis=1)
        den = jnp.concatenate([acc_ref[0, :, 128:130], acc_ref[1, :, 128:130]], axis=1)
        denr = jnp.broadcast_to(den[:, :, None], (n_pad, 4, 64)).reshape(n_pad, 256)
        h = num / denr + b_ref[...]
        if relu:
            h = jnp.maximum(h, 0.0)
        t = jnp.dot(h, w_ref[...], preferred_element_type=F32)
        t_ref[...] = jnp.concatenate(
            [t[:, 0:oc_next + 1], jnp.zeros((n_pad, wnext - oc_next - 1), F32)], axis=1)
        aux_ref[...] = jnp.concatenate(
            [t[:, oc_next + 1:oc_next + 2], jnp.zeros((n_pad, 15), F32)], axis=1)

    return _tc_call(body, [(n_pad, wnext), (n_pad, 16)])(acc, b.reshape(1, -1), wcat)


def _tc_norm1(acc, b, oc, n_pad):
    def body(acc_ref, b_ref, out_ref):
        num = acc_ref[0, :, 0:oc] + acc_ref[1, :, 0:oc]
        den = acc_ref[0, :, oc:oc + 1] + acc_ref[1, :, oc:oc + 1]
        out_ref[...] = num / den + b_ref[...]

    return _tc_call(body, [(n_pad, oc)])(acc, b.reshape(1, -1))[0]


def _tc_emp_proj4(emp_x_pad, df, wcat, n_pad):
    def body(x_ref, df_ref, wa_ref, wb_ref, t0_ref, t1_ref, aux_ref):
        t = (jnp.dot(x_ref[...], wa_ref[...], preferred_element_type=F32)
             + jnp.dot(df_ref[...], wb_ref[...], preferred_element_type=F32))
        z = jnp.zeros((n_pad, 6), F32)
        t0_ref[...] = jnp.concatenate([t[:, 0:128], t[:, 256:258], z], axis=1)
        t1_ref[...] = jnp.concatenate([t[:, 128:256], t[:, 258:260], z], axis=1)
        aux_ref[...] = jnp.concatenate(
            [t[:, 260:264], jnp.zeros((n_pad, 12), F32)], axis=1)

    return _tc_call(body, [(n_pad, 136), (n_pad, 136), (n_pad, 16)])(
        emp_x_pad, df, wcat[:128], wcat[128:])


def _tc_final(acc, b, n, n_pad):
    def body(acc_ref, b_ref, out_ref):
        num = acc_ref[0, :, 0:32] + acc_ref[1, :, 0:32]
        den = acc_ref[0, :, 32:33] + acc_ref[1, :, 32:33]
        x = num / den + b_ref[...]
        m = jnp.max(x, axis=1, keepdims=True)
        lse = m + jnp.log(jnp.sum(jnp.exp(x - m), axis=1, keepdims=True))
        out_ref[...] = (x - lse)[:n, :]

    return pl.pallas_call(
        body, out_shape=[jax.ShapeDtypeStruct((n, 32), F32)])(
            acc, b.reshape(1, -1))[0]


def _fold(W, a_s, a_d, heads, oc):
    eye = jnp.eye(heads, dtype=F32)
    A_s = (a_s[:, :, None] * eye[:, None, :]).reshape(heads * oc, heads)
    A_d = (a_d[:, :, None] * eye[:, None, :]).reshape(heads * oc, heads)
    return jnp.concatenate([W, W @ A_s, W @ A_d], axis=1)


def _pad_edges(edge_index, n, e_pad, c):
    pad = jnp.full((e_pad + c - edge_index.shape[1] - n,), n, I32)
    loop = jnp.arange(n, dtype=I32)
    return jnp.stack([
        jnp.concatenate([edge_index[0], loop, pad]),
        jnp.concatenate([edge_index[1], loop, pad])])


def kernel(dept_x, emp_x, dept_edge_index, emp_edge_index, dept_idx,
           W1, a_src1, a_dst1, b1, W2, a_src2, a_dst2, b2,
           W3, a_src3, a_dst3, b3, W4, a_src4, a_dst4, b4):
    Dn, En = dept_x.shape[0], emp_x.shape[0]
    Dnp = 1024
    Enp = 10240
    Dc, Ec = 64, 64
    De_pad = _round_up(dept_edge_index.shape[1] + Dn, NS * NC * Dc * 3)
    Ee_pad = _round_up(emp_edge_index.shape[1] + En, NS * NC * Ec * 3)

    dedges = _pad_edges(dept_edge_index, Dn, De_pad, Dc)
    eedges = _pad_edges(emp_edge_index, En, Ee_pad, Ec)
    dept_x_pad = jnp.pad(dept_x, ((0, Dnp - Dn), (0, 0)))
    emp_x_pad = jnp.pad(emp_x, ((0, Enp - En), (0, 0)))
    didx_pad = jnp.pad(dept_idx, (0, Enp - En))

    wc1 = _fold(W1, a_src1, a_dst1, 4, 64)
    wc2 = _fold(W2, a_src2, a_dst2, 1, 64)
    wc3 = _fold(W3, a_src3, a_dst3, 4, 64)
    wc4 = _fold(W4, a_src4, a_dst4, 1, 32)

    t0, t1, aux = _tc_proj4(dept_x_pad, wc1, Dnp)
    acc1 = _make_gat_sc(Dnp, De_pad, 4, 64, 2, Dc)(t0, t1, aux, dedges)
    t2, aux2 = _tc_norm4_proj(acc1, b1, wc2, Dnp, 64, relu=True)
    acc2 = _make_gat_sc(Dnp, De_pad, 1, 64, 1, Dc)(t2, t2, aux2, dedges)
    d2 = _tc_norm1(acc2, b2, 64, Dnp)
    df = _make_gather_rows(Dnp, 64, Enp)(d2, didx_pad)
    t0, t1, aux = _tc_emp_proj4(emp_x_pad, df, wc3, Enp)
    acc3 = _make_gat_sc(Enp, Ee_pad, 4, 64, 2, Ec)(t0, t1, aux, eedges)
    t4, aux4 = _tc_norm4_proj(acc3, b3, wc4, Enp, 32, relu=True)
    acc4 = _make_gat_sc(Enp, Ee_pad, 1, 32, 1, Ec)(t4, t4, aux4, eedges)
    return _tc_final(acc4, b4, En, Enp)
